# Initial kernel scaffold; baseline (speedup 1.0000x reference)
import jax
import jax.numpy as jnp
from jax.experimental import pallas as pl


def kernel(opt_tensor, tensor):
    raise NotImplementedError("write your pallas kernel here")


# speedup vs baseline: 126.6631x; 126.6631x over previous
import jax
import jax.numpy as jnp
from jax.experimental import pallas as pl
from jax.experimental.pallas import tpu as pltpu

_R = 512
_C = 1024


def _minmax_body(a_ref, b_ref, out_ref):
    i = pl.program_id(0)
    a = a_ref[...]
    b = b_ref[...]
    amin, amax = jnp.min(a), jnp.max(a)
    bmin, bmax = jnp.min(b), jnp.max(b)

    @pl.when(i == 0)
    def _init():
        out_ref[0] = amin
        out_ref[1] = amax
        out_ref[2] = bmin
        out_ref[3] = bmax

    @pl.when(i != 0)
    def _acc():
        out_ref[0] = jnp.minimum(out_ref[0], amin)
        out_ref[1] = jnp.maximum(out_ref[1], amax)
        out_ref[2] = jnp.minimum(out_ref[2], bmin)
        out_ref[3] = jnp.maximum(out_ref[3], bmax)


def _hist_body(ed_ref, a_ref, b_ref, out_ref, acc_ref):
    i = pl.program_id(0)
    nb = pl.num_programs(0)

    @pl.when(i == 0)
    def _init():
        for t in range(2):
            for j in range(9):
                acc_ref[t, j] = 0.0

    a = a_ref[...]
    b = b_ref[...]
    for j in range(9):
        acc_ref[0, j] += jnp.sum((a >= ed_ref[0, j]).astype(jnp.float32))
        acc_ref[1, j] += jnp.sum((b >= ed_ref[1, j]).astype(jnp.float32))

    @pl.when(i == nb - 1)
    def _fin():
        n = jnp.float32(nb * _R * _C)
        loss = jnp.float32(0.0)
        for b in range(10):
            if b == 0:
                ca = n - acc_ref[0, 0]
                cb = n - acc_ref[1, 0]
            elif b == 9:
                ca = acc_ref[0, 8]
                cb = acc_ref[1, 8]
            else:
                ca = acc_ref[0, b - 1] - acc_ref[0, b]
                cb = acc_ref[1, b - 1] - acc_ref[1, b]
            d = ca - cb
            loss = loss + d * d
        out_ref[0, 0] = loss / jnp.float32(10.0)


def kernel(opt_tensor, tensor):
    n = opt_tensor.shape[0]
    a2 = opt_tensor.reshape(n // _C, _C)
    b2 = tensor.reshape(n // _C, _C)
    nb = n // (_R * _C)

    mm = pl.pallas_call(
        _minmax_body,
        grid=(nb,),
        in_specs=[
            pl.BlockSpec((_R, _C), lambda i: (i, 0)),
            pl.BlockSpec((_R, _C), lambda i: (i, 0)),
        ],
        out_specs=pl.BlockSpec(memory_space=pltpu.SMEM),
        out_shape=jax.ShapeDtypeStruct((4,), jnp.float32),
    )(a2, b2)

    ea = jnp.linspace(mm[0], mm[1], 11)
    eb = jnp.linspace(mm[2], mm[3], 11)
    edges = jnp.stack([ea[1:10], eb[1:10]])

    loss = pl.pallas_call(
        _hist_body,
        grid=(nb,),
        in_specs=[
            pl.BlockSpec(memory_space=pltpu.SMEM),
            pl.BlockSpec((_R, _C), lambda i: (i, 0)),
            pl.BlockSpec((_R, _C), lambda i: (i, 0)),
        ],
        out_specs=pl.BlockSpec(memory_space=pltpu.SMEM),
        out_shape=jax.ShapeDtypeStruct((1, 1), jnp.float32),
        scratch_shapes=[pltpu.SMEM((2, 16), jnp.float32)],
    )(edges, a2, b2)
    return loss[0, 0]
